# baseline (speedup 1.0000x reference)
import jax
import jax.numpy as jnp
from jax.experimental import pallas as pl

N = 10000
HEADS = 6


def _selu_pallas(x):
    def body(x_ref, o_ref):
        v = x_ref[...]
        o_ref[...] = 1.0507009873554805 * jnp.where(
            v > 0, v, 1.6732632423543772 * (jnp.exp(jnp.minimum(v, 0.0)) - 1.0))
    return pl.pallas_call(
        body,
        out_shape=jax.ShapeDtypeStruct(x.shape, x.dtype),
    )(x)


def _deform_gat(coord, x, src, dst, bd_mask, W, b, a_s, a_d, cout):
    h = (x @ W + b).reshape(-1, HEADS, cout)
    alpha_src = jnp.sum(h * a_s[None, :, :], axis=-1)
    alpha_dst = jnp.sum(h * a_d[None, :, :], axis=-1)
    e = jax.nn.leaky_relu(alpha_src[src] + alpha_dst[dst], negative_slope=0.2)
    m = jax.ops.segment_max(e, dst, num_segments=N)
    m = jnp.where(jnp.isfinite(m), m, 0.0)
    ex = jnp.exp(e - m[dst])
    s = jax.ops.segment_sum(ex, dst, num_segments=N)
    alpha = ex / (s[dst] + 1e-16)
    agg = jax.ops.segment_sum(alpha[:, :, None] * h[src], dst, num_segments=N)
    out_feat = _selu_pallas(jnp.mean(agg, axis=1))
    am = jnp.mean(alpha, axis=1)
    new_coord = jax.ops.segment_sum(am[:, None] * coord[src], dst, num_segments=N)
    out_coord = jnp.where(bd_mask[:, None], coord, new_coord)
    return out_coord, out_feat


def kernel(data, edge_idx, bd_mask, poly_mesh, lin_w, lin_b, W1, b1, att_src1, att_dst1, W2, b2, att_src2, att_dst2, W3, b3, att_src3, att_dst3, W4, b4, att_src4, att_dst4):
    src = edge_idx[0]
    dst = edge_idx[1]
    coords = data[:, 0:2]
    lin1 = jax.nn.selu(data @ lin_w + lin_b)
    t1 = jnp.concatenate([coords, lin1], axis=1)
    c1, f1 = _deform_gat(coords, t1, src, dst, bd_mask, W1, b1, att_src1, att_dst1, 508)
    t2 = jnp.concatenate([c1, coords, f1], axis=1)
    c2, f2 = _deform_gat(c1, t2, src, dst, bd_mask, W2, b2, att_src2, att_dst2, 250)
    t3 = jnp.concatenate([c2, c1, coords, f2], axis=1)
    c3, f3 = _deform_gat(c2, t3, src, dst, bd_mask, W3, b3, att_src3, att_dst3, 120)
    t4 = jnp.concatenate([c3, c2, c1, coords, f3], axis=1)
    c4, f4 = _deform_gat(c3, t4, src, dst, bd_mask, W4, b4, att_src4, att_dst4, 20)
    return c4


# speedup vs baseline: 4.1920x; 4.1920x over previous
import functools

import jax
import jax.numpy as jnp
from jax import lax
from jax.experimental import pallas as pl

N = 10000
E = 160000
HEADS = 6
HP = 8


def _selu(v):
    return 1.0507009873554805 * jnp.where(
        v > 0, v, 1.6732632423543772 * (jnp.exp(jnp.minimum(v, 0.0)) - 1.0))


def _mm_selu(x, w, b):
    n, k = x.shape
    m = w.shape[1]
    bn = 1000

    def body(x_ref, w_ref, b_ref, o_ref):
        h = jnp.dot(x_ref[...], w_ref[...], preferred_element_type=jnp.float32)
        o_ref[...] = _selu(h + b_ref[...])

    return pl.pallas_call(
        body,
        grid=(n // bn,),
        in_specs=[
            pl.BlockSpec((bn, k), lambda i: (i, 0)),
            pl.BlockSpec((k, m), lambda i: (0, 0)),
            pl.BlockSpec((1, m), lambda i: (0, 0)),
        ],
        out_specs=pl.BlockSpec((bn, m), lambda i: (i, 0)),
        out_shape=jax.ShapeDtypeStruct((n, m), jnp.float32),
    )(x, w, b.reshape(1, m))


def _gat_dense(x, wp, bp, amat):
    n, k = x.shape
    m = wp.shape[1]
    bn = 1000

    def body(x_ref, w_ref, b_ref, a_ref, h_ref, asd_ref, mx_ref):
        i = pl.program_id(0)
        h = jnp.dot(x_ref[...], w_ref[...], preferred_element_type=jnp.float32)
        h = h + b_ref[...]
        h_ref[...] = h
        asd = jnp.dot(h, a_ref[...], preferred_element_type=jnp.float32)
        asd_ref[...] = asd

        @pl.when(i == 0)
        def _():
            mx_ref[...] = jnp.full((8, 128), -1e30, jnp.float32)

        mxb = jnp.max(asd, axis=0, keepdims=True)
        mx_ref[...] = jnp.maximum(mx_ref[...], jnp.broadcast_to(mxb, (8, 128)))

    return pl.pallas_call(
        body,
        grid=(n // bn,),
        in_specs=[
            pl.BlockSpec((bn, k), lambda i: (i, 0)),
            pl.BlockSpec((k, m), lambda i: (0, 0)),
            pl.BlockSpec((1, m), lambda i: (0, 0)),
            pl.BlockSpec((m, 128), lambda i: (0, 0)),
        ],
        out_specs=[
            pl.BlockSpec((bn, m), lambda i: (i, 0)),
            pl.BlockSpec((bn, 128), lambda i: (i, 0)),
            pl.BlockSpec((8, 128), lambda i: (0, 0)),
        ],
        out_shape=[
            jax.ShapeDtypeStruct((n, m), jnp.float32),
            jax.ShapeDtypeStruct((n, 128), jnp.float32),
            jax.ShapeDtypeStruct((8, 128), jnp.float32),
        ],
    )(x, wp, bp.reshape(1, m), amat)


def _combine(f0, f1, c0, c1, coordp, maskp):
    n, m = f0.shape
    bn = 2000

    def body(f0r, f1r, c0r, c1r, cpr, mkr, fo, co):
        fo[...] = _selu(f0r[...] + f1r[...])
        mk = mkr[...]
        co[...] = mk * cpr[...] + (1.0 - mk) * (c0r[...] + c1r[...])

    return pl.pallas_call(
        body,
        grid=(n // bn,),
        in_specs=[
            pl.BlockSpec((bn, m), lambda i: (i, 0)),
            pl.BlockSpec((bn, m), lambda i: (i, 0)),
            pl.BlockSpec((bn, 8), lambda i: (i, 0)),
            pl.BlockSpec((bn, 8), lambda i: (i, 0)),
            pl.BlockSpec((bn, 8), lambda i: (i, 0)),
            pl.BlockSpec((bn, 8), lambda i: (i, 0)),
        ],
        out_specs=[
            pl.BlockSpec((bn, m), lambda i: (i, 0)),
            pl.BlockSpec((bn, 8), lambda i: (i, 0)),
        ],
        out_shape=[
            jax.ShapeDtypeStruct((n, m), jnp.float32),
            jax.ShapeDtypeStruct((n, 8), jnp.float32),
        ],
    )(f0, f1, c0, c1, coordp, maskp)


def _edge_phase(h, asd, mvec, src, dst, coordp, copad):
    asrc = asd[:, :HP]
    adst = asd[:, HP:16]
    e = jax.nn.leaky_relu(asrc[src] + adst[dst], negative_slope=0.2)
    ex = jnp.exp(e - mvec[None, :HP])
    s = jax.ops.segment_sum(ex, dst, num_segments=N)
    rinv = 1.0 / (s + 1e-16)
    al6 = (ex * rinv[dst]) * (1.0 / HEADS)
    al6 = al6[:, :HEADS]
    hh = h.reshape(N, HEADS, copad)
    msg = jnp.sum(al6[:, :, None] * hh[src], axis=1)
    feat = jax.ops.segment_sum(msg, dst, num_segments=N)
    am = jnp.sum(al6, axis=1)
    csum = jax.ops.segment_sum(am[:, None] * coordp[src], dst, num_segments=N)
    return feat, csum


def _pad_layer_params(W, b, a_s, a_d, cin, cout, copad):
    hw = W.reshape(cin, HEADS, cout)
    wp = jnp.pad(hw, ((0, 0), (0, 0), (0, copad - cout))).reshape(cin, HEADS * copad)
    hb = b.reshape(HEADS, cout)
    bp = jnp.pad(hb, ((0, 0), (0, copad - cout))).reshape(HEADS * copad)
    asp = jnp.pad(a_s, ((0, 0), (0, copad - cout)))
    adp = jnp.pad(a_d, ((0, 0), (0, copad - cout)))
    eye = jnp.eye(HEADS, HP, dtype=jnp.float32)
    ablk_s = asp[:, :, None] * eye[:, None, :]
    ablk_d = adp[:, :, None] * eye[:, None, :]
    amat = jnp.concatenate([ablk_s, ablk_d], axis=2).reshape(HEADS * copad, 16)
    amat = jnp.pad(amat, ((0, 0), (0, 112)))
    return wp, bp, amat


def _layer(x, coordp, maskp, src, dst, W, b, a_s, a_d, cout, copad):
    cin = x.shape[1]
    wp, bp, amat = _pad_layer_params(W, b, a_s, a_d, cin, cout, copad)
    h, asd, mx = _gat_dense(x, wp, bp, amat)
    mvec = mx[0, :8] + mx[0, 8:16]
    feat, csum = _edge_phase(h, asd, mvec, src, dst, coordp, copad)
    featp, coordo = _combine(feat, jnp.zeros_like(feat), csum,
                             jnp.zeros_like(csum), coordp, maskp)
    return coordo, featp[:, :cout]


def kernel(data, edge_idx, bd_mask, poly_mesh, lin_w, lin_b, W1, b1, att_src1, att_dst1, W2, b2, att_src2, att_dst2, W3, b3, att_src3, att_dst3, W4, b4, att_src4, att_dst4):
    src = edge_idx[0]
    dst = edge_idx[1]
    coords = data[:, 0:2]
    coordp = jnp.pad(coords, ((0, 0), (0, 6)))
    maskp = jnp.broadcast_to(bd_mask.astype(jnp.float32)[:, None], (N, 8))

    lin1 = _mm_selu(data, lin_w, lin_b)
    t = jnp.concatenate([coords, lin1], axis=1)

    c1, f1 = _layer(t, coordp, maskp, src, dst, W1, b1, att_src1, att_dst1, 508, 512)
    t = jnp.concatenate([c1[:, :2], coords, f1], axis=1)
    c2, f2 = _layer(t, c1, maskp, src, dst, W2, b2, att_src2, att_dst2, 250, 256)
    t = jnp.concatenate([c2[:, :2], c1[:, :2], coords, f2], axis=1)
    c3, f3 = _layer(t, c2, maskp, src, dst, W3, b3, att_src3, att_dst3, 120, 128)
    t = jnp.concatenate([c3[:, :2], c2[:, :2], c1[:, :2], coords, f3], axis=1)
    c4, f4 = _layer(t, c3, maskp, src, dst, W4, b4, att_src4, att_dst4, 20, 128)
    return c4[:, :2]


# speedup vs baseline: 5.1950x; 1.2393x over previous
import functools

import jax
import jax.numpy as jnp
from jax import lax
from jax.experimental import pallas as pl
from jax.experimental.pallas import tpu as pltpu
from jax.experimental.pallas import tpu_sc as plsc

N = 10000
E = 160000
HEADS = 6
NPAD = 10240
EPAD = 163840
NTILES = 32
EPT = EPAD // NTILES
NB1 = EPT // 128
NB2 = EPT // 64
ROWS_PT = NPAD // 16

_SC_MESH = plsc.VectorSubcoreMesh(core_axis_name="c", subcore_axis_name="s",
                                  num_cores=2, num_subcores=16)


def _selu(v):
    return 1.0507009873554805 * jnp.where(
        v > 0, v, 1.6732632423543772 * (jnp.exp(jnp.minimum(v, 0.0)) - 1.0))


def _mm_selu(x, w, b):
    n, k = x.shape
    m = w.shape[1]
    bn = 1000

    def body(x_ref, w_ref, b_ref, o_ref):
        h = jnp.dot(x_ref[...], w_ref[...], preferred_element_type=jnp.float32)
        o_ref[...] = _selu(h + b_ref[...])

    return pl.pallas_call(
        body,
        grid=(n // bn,),
        in_specs=[
            pl.BlockSpec((bn, k), lambda i: (i, 0)),
            pl.BlockSpec((k, m), lambda i: (0, 0)),
            pl.BlockSpec((1, m), lambda i: (0, 0)),
        ],
        out_specs=pl.BlockSpec((bn, m), lambda i: (i, 0)),
        out_shape=jax.ShapeDtypeStruct((n, m), jnp.float32),
    )(x, w, b.reshape(1, m))


def _gat_dense(x, wp, bp, amat):
    n, k = x.shape
    m = wp.shape[1]
    bn = 1000

    def body(x_ref, w_ref, b_ref, a_ref, h_ref, asd_ref, mx_ref):
        i = pl.program_id(0)
        h = jnp.dot(x_ref[...], w_ref[...], preferred_element_type=jnp.float32)
        h = h + b_ref[...]
        h_ref[...] = h
        asd = jnp.dot(h, a_ref[...], preferred_element_type=jnp.float32)
        asd_ref[...] = asd

        @pl.when(i == 0)
        def _():
            mx_ref[...] = jnp.full((8, 128), -1e30, jnp.float32)

        mxb = jnp.max(asd, axis=0, keepdims=True)
        mx_ref[...] = jnp.maximum(mx_ref[...], jnp.broadcast_to(mxb, (8, 128)))

    return pl.pallas_call(
        body,
        grid=(n // bn,),
        in_specs=[
            pl.BlockSpec((bn, k), lambda i: (i, 0)),
            pl.BlockSpec((k, m), lambda i: (0, 0)),
            pl.BlockSpec((1, m), lambda i: (0, 0)),
            pl.BlockSpec((m, 128), lambda i: (0, 0)),
        ],
        out_specs=[
            pl.BlockSpec((bn, m), lambda i: (i, 0)),
            pl.BlockSpec((bn, 128), lambda i: (i, 0)),
            pl.BlockSpec((8, 128), lambda i: (0, 0)),
        ],
        out_shape=[
            jax.ShapeDtypeStruct((n, m), jnp.float32),
            jax.ShapeDtypeStruct((n, 128), jnp.float32),
            jax.ShapeDtypeStruct((8, 128), jnp.float32),
        ],
    )(x, wp, bp.reshape(1, m), amat)


def _recip(s0, s1):
    n, m = s0.shape
    bn = 2048

    def body(a_ref, b_ref, o_ref):
        o_ref[...] = (1.0 / HEADS) / (a_ref[...] + b_ref[...] + 1e-16)

    return pl.pallas_call(
        body,
        grid=(n // bn,),
        in_specs=[
            pl.BlockSpec((bn, m), lambda i: (i, 0)),
            pl.BlockSpec((bn, m), lambda i: (i, 0)),
        ],
        out_specs=pl.BlockSpec((bn, m), lambda i: (i, 0)),
        out_shape=jax.ShapeDtypeStruct((n, m), jnp.float32),
    )(s0, s1)


def _combine(f0, f1, c0, c1, coordp, maskp):
    n, m = f0.shape
    bn = 2000

    def body(f0r, f1r, c0r, c1r, cpr, mkr, fo, co):
        fo[...] = _selu(f0r[...] + f1r[...])
        mk = mkr[...]
        co[...] = mk * cpr[...] + (1.0 - mk) * (c0r[...] + c1r[...])

    return pl.pallas_call(
        body,
        grid=(n // bn,),
        in_specs=[
            pl.BlockSpec((bn, m), lambda i: (i, 0)),
            pl.BlockSpec((bn, m), lambda i: (i, 0)),
            pl.BlockSpec((bn, 16), lambda i: (i, 0)),
            pl.BlockSpec((bn, 16), lambda i: (i, 0)),
            pl.BlockSpec((bn, 16), lambda i: (i, 0)),
            pl.BlockSpec((bn, 16), lambda i: (i, 0)),
        ],
        out_specs=[
            pl.BlockSpec((bn, m), lambda i: (i, 0)),
            pl.BlockSpec((bn, 16), lambda i: (i, 0)),
        ],
        out_shape=[
            jax.ShapeDtypeStruct((n, m), jnp.float32),
            jax.ShapeDtypeStruct((n, 16), jnp.float32),
        ],
    )(f0, f1, c0, c1, coordp, maskp)


def _tile_id():
    return lax.axis_index("c") * 16 + lax.axis_index("s")


def _zero_rows(zb, shared, srows):
    nz = zb.shape[0]
    for k2 in range(ROWS_PT // nz):
        pltpu.sync_copy(zb, shared.at[pl.ds(srows + k2 * nz, nz)])


NB1B = EPT // 64


def _zero_high_cols(buf):
    def zrow(i, _):
        for g in range(1, 8):
            buf[i, pl.ds(g * 16, 16)] = jnp.zeros((16,), jnp.float32)
        return 0

    lax.fori_loop(0, buf.shape[0], zrow, 0)


def _sc_scores(asrc_d, adst_d, mvec, src3, dst3, zf):
    def body(asrc_ref, adst_ref, mv_ref, src_ref, dst_ref, z_ref,
             ex_out, s_out, sbuf, dbuf, ag, dg, exw, ex16, mvb, s_sh,
             sema, semb):
        core = lax.axis_index("c")
        sub = lax.axis_index("s")
        wid = _tile_id()
        base = wid * NB1B
        pltpu.sync_copy(mv_ref, mvb)
        pltpu.sync_copy(z_ref, s_sh.at[pl.ds(sub * ROWS_PT, ROWS_PT)])
        _zero_high_cols(exw)
        plsc.subcore_barrier()

        mvv = mvb[...]

        def batch(j, _):
            pltpu.sync_copy(src_ref.at[pl.ds(base + j, 1)], sbuf)
            pltpu.sync_copy(dst_ref.at[pl.ds(base + j, 1)], dbuf)
            da = pltpu.async_copy(asrc_ref.at[sbuf.at[0]], ag, sema)
            db = pltpu.async_copy(adst_ref.at[dbuf.at[0]], dg, semb)
            da.wait()
            db.wait()

            def row(i, _):
                s16 = pl.ds(0, 16)
                e = ag[i, s16] + dg[i, s16]
                e = jnp.where(e > 0, e, 0.2 * e)
                v = jnp.exp(e - mvv)
                exw[i, s16] = v
                ex16[i, :] = v
                return 0

            lax.fori_loop(0, 64, row, 0)
            pltpu.sync_copy(ex16, ex_out.at[base + j])
            pltpu.sync_copy(exw, s_sh.at[dbuf.at[0]], add=True)
            return 0

        lax.fori_loop(0, NB1B, batch, 0)
        plsc.subcore_barrier()
        pltpu.sync_copy(s_sh.at[pl.ds(sub * ROWS_PT, ROWS_PT)],
                        s_out.at[core, pl.ds(sub * ROWS_PT, ROWS_PT)])

    return pl.kernel(
        body,
        out_type=[
            jax.ShapeDtypeStruct((EPAD // 64, 64, 16), jnp.float32),
            jax.ShapeDtypeStruct((2, NPAD, 128), jnp.float32),
        ],
        mesh=_SC_MESH,
        scratch_types=[
            pltpu.VMEM((1, 64), jnp.int32),
            pltpu.VMEM((1, 64), jnp.int32),
            pltpu.VMEM((64, 128), jnp.float32),
            pltpu.VMEM((64, 128), jnp.float32),
            pltpu.VMEM((64, 128), jnp.float32),
            pltpu.VMEM((64, 16), jnp.float32),
            pltpu.VMEM((16,), jnp.float32),
            pltpu.VMEM_SHARED((NPAD, 128), jnp.float32),
            pltpu.SemaphoreType.DMA,
            pltpu.SemaphoreType.DMA,
        ],
    )(asrc_d, adst_d, mvec, src3, dst3, zf)


def _sc_coords(ex, rinv, coord_g, src3, dst3, zf):
    def body(ex_ref, rinv_ref, coord_ref, src_ref, dst_ref, z_ref,
             c_out, sbuf, dbuf, exg, rig, cog, cmw, c_sh, sema, semb):
        core = lax.axis_index("c")
        sub = lax.axis_index("s")
        wid = _tile_id()
        base = wid * NB1B
        pltpu.sync_copy(z_ref, c_sh.at[pl.ds(sub * ROWS_PT, ROWS_PT)])
        _zero_high_cols(cmw)
        plsc.subcore_barrier()

        def batch(j, _):
            pltpu.sync_copy(src_ref.at[pl.ds(base + j, 1)], sbuf)
            pltpu.sync_copy(dst_ref.at[pl.ds(base + j, 1)], dbuf)
            pltpu.sync_copy(ex_ref.at[base + j], exg)
            da = pltpu.async_copy(rinv_ref.at[dbuf.at[0]], rig, sema)
            db = pltpu.async_copy(coord_ref.at[sbuf.at[0]], cog, semb)
            da.wait()
            db.wait()

            def row(i, _):
                s16 = pl.ds(0, 16)
                av = exg[i, :] * rig[i, s16]
                am = av[0] + av[1] + av[2] + av[3] + av[4] + av[5]
                cmw[i, s16] = am * cog[i, s16]
                return 0

            lax.fori_loop(0, 64, row, 0)
            pltpu.sync_copy(cmw, c_sh.at[dbuf.at[0]], add=True)
            return 0

        lax.fori_loop(0, NB1B, batch, 0)
        plsc.subcore_barrier()
        pltpu.sync_copy(c_sh.at[pl.ds(sub * ROWS_PT, ROWS_PT)],
                        c_out.at[core, pl.ds(sub * ROWS_PT, ROWS_PT)])

    return pl.kernel(
        body,
        out_type=jax.ShapeDtypeStruct((2, NPAD, 128), jnp.float32),
        mesh=_SC_MESH,
        scratch_types=[
            pltpu.VMEM((1, 64), jnp.int32),
            pltpu.VMEM((1, 64), jnp.int32),
            pltpu.VMEM((64, 16), jnp.float32),
            pltpu.VMEM((64, 128), jnp.float32),
            pltpu.VMEM((64, 128), jnp.float32),
            pltpu.VMEM((64, 128), jnp.float32),
            pltpu.VMEM_SHARED((NPAD, 128), jnp.float32),
            pltpu.SemaphoreType.DMA,
            pltpu.SemaphoreType.DMA,
        ],
    )(ex, rinv, coord_g, src3, dst3, zf)


NB2 = EPT // 32


def _make_sc_agg(nc):
    def body(hrows, ex_ref, rinv_ref, src_ref, dst_ref, z_ref,
             f_out, sbuf, dbuf, idx2, rows, exg, rig, msg, f_sh,
             gsem, sem2):
        core = lax.axis_index("c")
        sub = lax.axis_index("s")
        wid = _tile_id()
        base = wid * NB2
        ebase = wid * EPT

        for ci in range(nc):
            pltpu.sync_copy(z_ref, f_sh.at[pl.ds(sub * ROWS_PT, ROWS_PT)])
            plsc.subcore_barrier()

            def batch(j, _):
                pltpu.sync_copy(src_ref.at[pl.ds(base + j, 1)], sbuf)
                pltpu.sync_copy(dst_ref.at[pl.ds(base + j, 1)], dbuf)
                pltpu.sync_copy(ex_ref.at[pl.ds(ebase + j * 32, 32)], exg)
                pltpu.async_copy(rinv_ref.at[dbuf.at[0]], rig, sem2).wait()

                def grp(g, _):
                    sv = sbuf[0, pl.ds(g * 16, 16)]
                    for hh in range(HEADS):
                        idx2[hh, pl.ds(g * 16, 16)] = (
                            sv * (HEADS * nc) + (hh * nc + ci))
                    return 0

                lax.fori_loop(0, 2, grp, 0)
                dss = [pltpu.async_copy(hrows.at[idx2.at[hh]], rows.at[hh],
                                        gsem)
                       for hh in range(HEADS)]
                for d in dss:
                    d.wait()

                def erow(i, _):
                    av = exg[i, :] * rig[i, pl.ds(0, 16)]
                    a0 = av[0]
                    a1 = av[1]
                    a2 = av[2]
                    a3 = av[3]
                    a4 = av[4]
                    a5 = av[5]
                    for g in range(8):
                        sl = pl.ds(g * 16, 16)
                        v = a0 * rows[0, i, sl]
                        v = v + a1 * rows[1, i, sl]
                        v = v + a2 * rows[2, i, sl]
                        v = v + a3 * rows[3, i, sl]
                        v = v + a4 * rows[4, i, sl]
                        v = v + a5 * rows[5, i, sl]
                        msg[i, sl] = v
                    return 0

                lax.fori_loop(0, 32, erow, 0)
                pltpu.sync_copy(msg, f_sh.at[dbuf.at[0]], add=True)
                return 0

            lax.fori_loop(0, NB2, batch, 0)
            plsc.subcore_barrier()
            pltpu.sync_copy(
                f_sh.at[pl.ds(sub * ROWS_PT, ROWS_PT)],
                f_out.at[core, ci, pl.ds(sub * ROWS_PT, ROWS_PT)])
            plsc.subcore_barrier()

    return pl.kernel(
        body,
        out_type=jax.ShapeDtypeStruct((2, nc, NPAD, 128), jnp.float32),
        mesh=_SC_MESH,
        scratch_types=[
            pltpu.VMEM((1, 32), jnp.int32),
            pltpu.VMEM((1, 32), jnp.int32),
            pltpu.VMEM((HEADS, 32), jnp.int32),
            pltpu.VMEM((HEADS, 32, 128), jnp.float32),
            pltpu.VMEM((32, 16), jnp.float32),
            pltpu.VMEM((32, 128), jnp.float32),
            pltpu.VMEM((32, 128), jnp.float32),
            pltpu.VMEM_SHARED((NPAD, 128), jnp.float32),
            pltpu.SemaphoreType.DMA,
            pltpu.SemaphoreType.DMA,
        ],
    )


_SC_AGG = {1: _make_sc_agg(1), 2: _make_sc_agg(2), 4: _make_sc_agg(4)}


def _pad_layer_params(W, b, a_s, a_d, cin, cout, copad):
    hw = W.reshape(cin, HEADS, cout)
    wp = jnp.pad(hw, ((0, 0), (0, 0), (0, copad - cout))).reshape(cin, HEADS * copad)
    hb = b.reshape(HEADS, cout)
    bp = jnp.pad(hb, ((0, 0), (0, copad - cout))).reshape(HEADS * copad)
    asp = jnp.pad(a_s, ((0, 0), (0, copad - cout)))
    adp = jnp.pad(a_d, ((0, 0), (0, copad - cout)))
    eye = jnp.eye(HEADS, 8, dtype=jnp.float32)
    ablk_s = asp[:, :, None] * eye[:, None, :]
    ablk_d = adp[:, :, None] * eye[:, None, :]
    amat = jnp.concatenate([ablk_s, ablk_d], axis=2).reshape(HEADS * copad, 16)
    amat = jnp.pad(amat, ((0, 0), (0, 112)))
    return wp, bp, amat


def _layer(x, coordp, maskp, src3, dst3, src2c, dst2c, zf,
           W, b, a_s, a_d, cout, copad):
    cin = x.shape[1]
    wp, bp, amat = _pad_layer_params(W, b, a_s, a_d, cin, cout, copad)
    h, asd, mx = _gat_dense(x, wp, bp, amat)
    m8 = mx[0, :8] + mx[0, 8:16]
    mvec = jnp.concatenate([m8, m8])
    asrc_d = jnp.pad(jnp.concatenate([asd[:, :8], asd[:, :8]], axis=1),
                     ((0, NPAD - N), (0, 112)))
    adst_d = jnp.pad(jnp.concatenate([asd[:, 8:16], asd[:, 8:16]], axis=1),
                     ((0, NPAD - N), (0, 112)))
    coord_g = jnp.pad(coordp, ((0, NPAD - N), (0, 112)))

    ex, spart = _sc_scores(asrc_d, adst_d, mvec, src3, dst3, zf)
    rinv = jnp.pad(_recip(spart[0, :, :16], spart[1, :, :16]),
                   ((0, 0), (0, 112)))

    cpart = _sc_coords(ex, rinv, coord_g, src3, dst3, zf)
    nc = copad // 128
    hrows = h.reshape(N * HEADS * nc, 128)
    fpart = _SC_AGG[nc](hrows, ex.reshape(EPAD, 16), rinv,
                        src2c, dst2c, zf)
    f2 = fpart.transpose(0, 2, 1, 3).reshape(2, NPAD, copad)[:, :N]
    c2 = cpart[:, :N, :16]
    featp, coordo = _combine(f2[0], f2[1], c2[0], c2[1], coordp, maskp)
    return coordo, featp[:, :cout]


def kernel(data, edge_idx, bd_mask, poly_mesh, lin_w, lin_b, W1, b1, att_src1, att_dst1, W2, b2, att_src2, att_dst2, W3, b3, att_src3, att_dst3, W4, b4, att_src4, att_dst4):
    src = edge_idx[0]
    dst = edge_idx[1]
    srcp = jnp.concatenate([src, jnp.zeros((EPAD - E,), jnp.int32)])
    dstp = jnp.concatenate([dst, jnp.full((EPAD - E,), NPAD - 1, jnp.int32)])
    src3 = srcp.reshape(EPAD // 64, 64)
    dst3 = dstp.reshape(EPAD // 64, 64)
    src2c = srcp.reshape(EPAD // 32, 32)
    dst2c = dstp.reshape(EPAD // 32, 32)
    zf = jnp.zeros((ROWS_PT, 128), jnp.float32)

    coords = data[:, 0:2]
    coordp = jnp.pad(coords, ((0, 0), (0, 14)))
    maskp = jnp.broadcast_to(bd_mask.astype(jnp.float32)[:, None], (N, 16))

    lin1 = _mm_selu(data, lin_w, lin_b)
    t = jnp.concatenate([coords, lin1], axis=1)

    c1, f1 = _layer(t, coordp, maskp, src3, dst3, src2c, dst2c, zf,
                    W1, b1, att_src1, att_dst1, 508, 512)
    t = jnp.concatenate([c1[:, :2], coords, f1], axis=1)
    c2, f2 = _layer(t, c1, maskp, src3, dst3, src2c, dst2c, zf,
                    W2, b2, att_src2, att_dst2, 250, 256)
    t = jnp.concatenate([c2[:, :2], c1[:, :2], coords, f2], axis=1)
    c3, f3 = _layer(t, c2, maskp, src3, dst3, src2c, dst2c, zf,
                    W3, b3, att_src3, att_dst3, 120, 128)
    t = jnp.concatenate([c3[:, :2], c2[:, :2], c1[:, :2], coords, f3], axis=1)
    c4, f4 = _layer(t, c3, maskp, src3, dst3, src2c, dst2c, zf,
                    W4, b4, att_src4, att_dst4, 20, 128)
    return c4[:, :2]


# speedup vs baseline: 8.1487x; 1.5686x over previous
import functools

import jax
import jax.numpy as jnp
from jax import lax
from jax.experimental import pallas as pl
from jax.experimental.pallas import tpu as pltpu
from jax.experimental.pallas import tpu_sc as plsc

N = 10000
E = 160000
HEADS = 6
NPAD = 10240
EPAD = 163840
NTILES = 32
EPT = EPAD // NTILES
NB1 = EPT // 128
NB2 = EPT // 64
ROWS_PT = NPAD // 16

_SC_MESH = plsc.VectorSubcoreMesh(core_axis_name="c", subcore_axis_name="s",
                                  num_cores=2, num_subcores=16)


def _selu(v):
    return 1.0507009873554805 * jnp.where(
        v > 0, v, 1.6732632423543772 * (jnp.exp(jnp.minimum(v, 0.0)) - 1.0))


def _mm_selu(x, w, b):
    n, k = x.shape
    m = w.shape[1]
    bn = 1000

    def body(x_ref, w_ref, b_ref, o_ref):
        h = jnp.dot(x_ref[...], w_ref[...], preferred_element_type=jnp.float32)
        o_ref[...] = _selu(h + b_ref[...])

    return pl.pallas_call(
        body,
        grid=(n // bn,),
        in_specs=[
            pl.BlockSpec((bn, k), lambda i: (i, 0)),
            pl.BlockSpec((k, m), lambda i: (0, 0)),
            pl.BlockSpec((1, m), lambda i: (0, 0)),
        ],
        out_specs=pl.BlockSpec((bn, m), lambda i: (i, 0)),
        out_shape=jax.ShapeDtypeStruct((n, m), jnp.float32),
    )(x, w, b.reshape(1, m))


def _gat_dense(x, wp, bp, amat):
    n, k = x.shape
    m = wp.shape[1]
    bn = 1000

    def body(x_ref, w_ref, b_ref, a_ref, h_ref, asd_ref, mx_ref):
        i = pl.program_id(0)
        h = jnp.dot(x_ref[...], w_ref[...], preferred_element_type=jnp.float32)
        h = h + b_ref[...]
        h_ref[...] = h
        asd = jnp.dot(h, a_ref[...], preferred_element_type=jnp.float32)
        asd_ref[...] = asd

        @pl.when(i == 0)
        def _():
            mx_ref[...] = jnp.full((8, 128), -1e30, jnp.float32)

        mxb = jnp.max(asd, axis=0, keepdims=True)
        mx_ref[...] = jnp.maximum(mx_ref[...], jnp.broadcast_to(mxb, (8, 128)))

    return pl.pallas_call(
        body,
        grid=(n // bn,),
        in_specs=[
            pl.BlockSpec((bn, k), lambda i: (i, 0)),
            pl.BlockSpec((k, m), lambda i: (0, 0)),
            pl.BlockSpec((1, m), lambda i: (0, 0)),
            pl.BlockSpec((m, 128), lambda i: (0, 0)),
        ],
        out_specs=[
            pl.BlockSpec((bn, m), lambda i: (i, 0)),
            pl.BlockSpec((bn, 128), lambda i: (i, 0)),
            pl.BlockSpec((8, 128), lambda i: (0, 0)),
        ],
        out_shape=[
            jax.ShapeDtypeStruct((n, m), jnp.float32),
            jax.ShapeDtypeStruct((n, 128), jnp.float32),
            jax.ShapeDtypeStruct((8, 128), jnp.float32),
        ],
    )(x, wp, bp.reshape(1, m), amat)


def _recip(s0, s1):
    n, m = s0.shape
    bn = 2048

    def body(a_ref, b_ref, o_ref):
        o_ref[...] = (1.0 / HEADS) / (a_ref[...] + b_ref[...] + 1e-16)

    return pl.pallas_call(
        body,
        grid=(n // bn,),
        in_specs=[
            pl.BlockSpec((bn, m), lambda i: (i, 0)),
            pl.BlockSpec((bn, m), lambda i: (i, 0)),
        ],
        out_specs=pl.BlockSpec((bn, m), lambda i: (i, 0)),
        out_shape=jax.ShapeDtypeStruct((n, m), jnp.float32),
    )(s0, s1)


def _combine(f0, f1, c0, c1, coordp, maskp):
    n, m = f0.shape
    bn = 2000

    def body(f0r, f1r, c0r, c1r, cpr, mkr, fo, co):
        fo[...] = _selu(f0r[...] + f1r[...])
        mk = mkr[...]
        co[...] = mk * cpr[...] + (1.0 - mk) * (c0r[...] + c1r[...])

    return pl.pallas_call(
        body,
        grid=(n // bn,),
        in_specs=[
            pl.BlockSpec((bn, m), lambda i: (i, 0)),
            pl.BlockSpec((bn, m), lambda i: (i, 0)),
            pl.BlockSpec((bn, 16), lambda i: (i, 0)),
            pl.BlockSpec((bn, 16), lambda i: (i, 0)),
            pl.BlockSpec((bn, 16), lambda i: (i, 0)),
            pl.BlockSpec((bn, 16), lambda i: (i, 0)),
        ],
        out_specs=[
            pl.BlockSpec((bn, m), lambda i: (i, 0)),
            pl.BlockSpec((bn, 16), lambda i: (i, 0)),
        ],
        out_shape=[
            jax.ShapeDtypeStruct((n, m), jnp.float32),
            jax.ShapeDtypeStruct((n, 16), jnp.float32),
        ],
    )(f0, f1, c0, c1, coordp, maskp)


def _tile_id():
    return lax.axis_index("c") * 16 + lax.axis_index("s")


def _zero_rows(zb, shared, srows):
    nz = zb.shape[0]
    for k2 in range(ROWS_PT // nz):
        pltpu.sync_copy(zb, shared.at[pl.ds(srows + k2 * nz, nz)])


NB1B = EPT // 64


def _zero_high_cols(buf):
    def zrow(i, _):
        for g in range(1, 8):
            buf[i, pl.ds(g * 16, 16)] = jnp.zeros((16,), jnp.float32)
        return 0

    lax.fori_loop(0, buf.shape[0], zrow, 0)


def _sc_scores(asrc_d, adst_d, mvec, src3, dst3, zf):
    def body(asrc_ref, adst_ref, mv_ref, src_ref, dst_ref, z_ref,
             ex_out, s_out, sbuf, dbuf, ag, dg, exw, ex16, mvb, s_sh,
             sema, semb):
        core = lax.axis_index("c")
        sub = lax.axis_index("s")
        wid = _tile_id()
        base = wid * NB1B
        pltpu.sync_copy(mv_ref, mvb)
        pltpu.sync_copy(z_ref, s_sh.at[pl.ds(sub * ROWS_PT, ROWS_PT)])
        _zero_high_cols(exw)
        plsc.subcore_barrier()

        mvv = mvb[...]

        def batch(j, _):
            pltpu.sync_copy(src_ref.at[pl.ds(base + j, 1)], sbuf)
            pltpu.sync_copy(dst_ref.at[pl.ds(base + j, 1)], dbuf)
            da = pltpu.async_copy(asrc_ref.at[sbuf.at[0]], ag, sema)
            db = pltpu.async_copy(adst_ref.at[dbuf.at[0]], dg, semb)
            da.wait()
            db.wait()

            def row(i, _):
                s16 = pl.ds(0, 16)
                e = ag[i, s16] + dg[i, s16]
                e = jnp.where(e > 0, e, 0.2 * e)
                v = jnp.exp(e - mvv)
                exw[i, s16] = v
                ex16[i, :] = v
                return 0

            lax.fori_loop(0, 64, row, 0)
            pltpu.sync_copy(ex16, ex_out.at[base + j])
            pltpu.sync_copy(exw, s_sh.at[dbuf.at[0]], add=True)
            return 0

        lax.fori_loop(0, NB1B, batch, 0)
        plsc.subcore_barrier()
        pltpu.sync_copy(s_sh.at[pl.ds(sub * ROWS_PT, ROWS_PT)],
                        s_out.at[core, pl.ds(sub * ROWS_PT, ROWS_PT)])

    return pl.kernel(
        body,
        out_type=[
            jax.ShapeDtypeStruct((EPAD // 64, 64, 16), jnp.float32),
            jax.ShapeDtypeStruct((2, NPAD, 128), jnp.float32),
        ],
        mesh=_SC_MESH,
        scratch_types=[
            pltpu.VMEM((1, 64), jnp.int32),
            pltpu.VMEM((1, 64), jnp.int32),
            pltpu.VMEM((64, 128), jnp.float32),
            pltpu.VMEM((64, 128), jnp.float32),
            pltpu.VMEM((64, 128), jnp.float32),
            pltpu.VMEM((64, 16), jnp.float32),
            pltpu.VMEM((16,), jnp.float32),
            pltpu.VMEM_SHARED((NPAD, 128), jnp.float32),
            pltpu.SemaphoreType.DMA,
            pltpu.SemaphoreType.DMA,
        ],
    )(asrc_d, adst_d, mvec, src3, dst3, zf)


def _sc_coords(ex, rinv, coord_g, src3, dst3, zf):
    def body(ex_ref, rinv_ref, coord_ref, src_ref, dst_ref, z_ref,
             c_out, sbuf, dbuf, exg, rig, cog, cmw, c_sh, sema, semb):
        core = lax.axis_index("c")
        sub = lax.axis_index("s")
        wid = _tile_id()
        base = wid * NB1B
        pltpu.sync_copy(z_ref, c_sh.at[pl.ds(sub * ROWS_PT, ROWS_PT)])
        _zero_high_cols(cmw)
        plsc.subcore_barrier()

        def batch(j, _):
            pltpu.sync_copy(src_ref.at[pl.ds(base + j, 1)], sbuf)
            pltpu.sync_copy(dst_ref.at[pl.ds(base + j, 1)], dbuf)
            pltpu.sync_copy(ex_ref.at[base + j], exg)
            da = pltpu.async_copy(rinv_ref.at[dbuf.at[0]], rig, sema)
            db = pltpu.async_copy(coord_ref.at[sbuf.at[0]], cog, semb)
            da.wait()
            db.wait()

            def row(i, _):
                s16 = pl.ds(0, 16)
                av = exg[i, :] * rig[i, s16]
                am = av[0] + av[1] + av[2] + av[3] + av[4] + av[5]
                cmw[i, s16] = am * cog[i, s16]
                return 0

            lax.fori_loop(0, 64, row, 0)
            pltpu.sync_copy(cmw, c_sh.at[dbuf.at[0]], add=True)
            return 0

        lax.fori_loop(0, NB1B, batch, 0)
        plsc.subcore_barrier()
        pltpu.sync_copy(c_sh.at[pl.ds(sub * ROWS_PT, ROWS_PT)],
                        c_out.at[core, pl.ds(sub * ROWS_PT, ROWS_PT)])

    return pl.kernel(
        body,
        out_type=jax.ShapeDtypeStruct((2, NPAD, 128), jnp.float32),
        mesh=_SC_MESH,
        scratch_types=[
            pltpu.VMEM((1, 64), jnp.int32),
            pltpu.VMEM((1, 64), jnp.int32),
            pltpu.VMEM((64, 16), jnp.float32),
            pltpu.VMEM((64, 128), jnp.float32),
            pltpu.VMEM((64, 128), jnp.float32),
            pltpu.VMEM((64, 128), jnp.float32),
            pltpu.VMEM_SHARED((NPAD, 128), jnp.float32),
            pltpu.SemaphoreType.DMA,
            pltpu.SemaphoreType.DMA,
        ],
    )(ex, rinv, coord_g, src3, dst3, zf)


NB2 = EPT // 16
NI2 = NB2 // 2


def _make_sc_agg(nc):
    def body(hrows, ex_ref, rinv_ref, src_ref, dst_ref, z_ref,
             f_out, srcb, dstb, exga, riga, rowsa,
             exgb, rigb, rowsb, msg, f_sh, gsema, gsemb):
        core = lax.axis_index("c")
        sub = lax.axis_index("s")
        wid = _tile_id()
        pbase = wid * (EPT // 128)
        ebase = wid * EPT
        pltpu.sync_copy(src_ref.at[pl.ds(pbase, EPT // 128)], srcb)
        pltpu.sync_copy(dst_ref.at[pl.ds(pbase, EPT // 128)], dstb)

        def edge_vecs(bid):
            r = bid // 8
            co = (bid % 8) * 16
            return srcb[r, pl.ds(co, 16)], dstb[r, pl.ds(co, 16)]

        def fire(bid, exgx, rigx, rowsx, gsemx, ci):
            sv, dv = edge_vecs(bid)
            pltpu.sync_copy(ex_ref.at[pl.ds(ebase + bid * 16, 16)], exgx)
            pltpu.async_copy(rinv_ref.at[dv], rigx, gsemx)
            for hh in range(HEADS):
                iv = sv * (HEADS * nc) + (hh * nc + ci)
                pltpu.async_copy(hrows.at[iv], rowsx.at[hh], gsemx)

        def drain(rigx, rowsx, gsemx):
            pltpu.make_async_copy(rinv_ref.at[pl.ds(0, 16)], rigx,
                                  gsemx).wait()
            for hh in range(HEADS):
                pltpu.make_async_copy(hrows.at[pl.ds(0, 16)], rowsx.at[hh],
                                      gsemx).wait()

        def compute(bid, exgx, rigx, rowsx):
            def erow(i, _):
                av = exgx[i, :] * rigx[i, pl.ds(0, 16)]
                a0 = av[0]
                a1 = av[1]
                a2 = av[2]
                a3 = av[3]
                a4 = av[4]
                a5 = av[5]
                for g in range(8):
                    sl = pl.ds(g * 16, 16)
                    v = a0 * rowsx[0, i, sl]
                    v = v + a1 * rowsx[1, i, sl]
                    v = v + a2 * rowsx[2, i, sl]
                    v = v + a3 * rowsx[3, i, sl]
                    v = v + a4 * rowsx[4, i, sl]
                    v = v + a5 * rowsx[5, i, sl]
                    msg[i, sl] = v
                return 0

            lax.fori_loop(0, 16, erow, 0)
            _, dv = edge_vecs(bid)
            pltpu.sync_copy(msg, f_sh.at[dv], add=True)

        for ci in range(nc):
            pltpu.sync_copy(z_ref, f_sh.at[pl.ds(sub * ROWS_PT, ROWS_PT)])
            plsc.subcore_barrier()
            fire(0, exga, riga, rowsa, gsema, ci)

            def it(t, _):
                fire(2 * t + 1, exgb, rigb, rowsb, gsemb, ci)
                drain(riga, rowsa, gsema)
                compute(2 * t, exga, riga, rowsa)

                @pl.when(t < NI2 - 1)
                def _():
                    fire(2 * t + 2, exga, riga, rowsa, gsema, ci)

                drain(rigb, rowsb, gsemb)
                compute(2 * t + 1, exgb, rigb, rowsb)
                return 0

            lax.fori_loop(0, NI2, it, 0)
            plsc.subcore_barrier()
            pltpu.sync_copy(
                f_sh.at[pl.ds(sub * ROWS_PT, ROWS_PT)],
                f_out.at[core, ci, pl.ds(sub * ROWS_PT, ROWS_PT)])
            plsc.subcore_barrier()

    return pl.kernel(
        body,
        out_type=jax.ShapeDtypeStruct((2, nc, NPAD, 128), jnp.float32),
        mesh=_SC_MESH,
        scratch_types=[
            pltpu.VMEM((EPT // 128, 128), jnp.int32),
            pltpu.VMEM((EPT // 128, 128), jnp.int32),
            pltpu.VMEM((16, 16), jnp.float32),
            pltpu.VMEM((16, 128), jnp.float32),
            pltpu.VMEM((HEADS, 16, 128), jnp.float32),
            pltpu.VMEM((16, 16), jnp.float32),
            pltpu.VMEM((16, 128), jnp.float32),
            pltpu.VMEM((HEADS, 16, 128), jnp.float32),
            pltpu.VMEM((16, 128), jnp.float32),
            pltpu.VMEM_SHARED((NPAD, 128), jnp.float32),
            pltpu.SemaphoreType.DMA,
            pltpu.SemaphoreType.DMA,
        ],
    )


_SC_AGG = {1: _make_sc_agg(1), 2: _make_sc_agg(2), 4: _make_sc_agg(4)}


def _pad_layer_params(W, b, a_s, a_d, cin, cout, copad):
    hw = W.reshape(cin, HEADS, cout)
    wp = jnp.pad(hw, ((0, 0), (0, 0), (0, copad - cout))).reshape(cin, HEADS * copad)
    hb = b.reshape(HEADS, cout)
    bp = jnp.pad(hb, ((0, 0), (0, copad - cout))).reshape(HEADS * copad)
    asp = jnp.pad(a_s, ((0, 0), (0, copad - cout)))
    adp = jnp.pad(a_d, ((0, 0), (0, copad - cout)))
    eye = jnp.eye(HEADS, 8, dtype=jnp.float32)
    ablk_s = asp[:, :, None] * eye[:, None, :]
    ablk_d = adp[:, :, None] * eye[:, None, :]
    amat = jnp.concatenate([ablk_s, ablk_d], axis=2).reshape(HEADS * copad, 16)
    amat = jnp.pad(amat, ((0, 0), (0, 112)))
    return wp, bp, amat


def _layer(x, coordp, maskp, src3, dst3, src2c, dst2c, zf,
           W, b, a_s, a_d, cout, copad):
    cin = x.shape[1]
    wp, bp, amat = _pad_layer_params(W, b, a_s, a_d, cin, cout, copad)
    h, asd, mx = _gat_dense(x, wp, bp, amat)
    m8 = mx[0, :8] + mx[0, 8:16]
    mvec = jnp.concatenate([m8, m8])
    asrc_d = jnp.pad(jnp.concatenate([asd[:, :8], asd[:, :8]], axis=1),
                     ((0, NPAD - N), (0, 112)))
    adst_d = jnp.pad(jnp.concatenate([asd[:, 8:16], asd[:, 8:16]], axis=1),
                     ((0, NPAD - N), (0, 112)))
    coord_g = jnp.pad(coordp, ((0, NPAD - N), (0, 112)))

    ex, spart = _sc_scores(asrc_d, adst_d, mvec, src3, dst3, zf)
    rinv = jnp.pad(_recip(spart[0, :, :16], spart[1, :, :16]),
                   ((0, 0), (0, 112)))

    cpart = _sc_coords(ex, rinv, coord_g, src3, dst3, zf)
    nc = copad // 128
    hrows = h.reshape(N * HEADS * nc, 128)
    fpart = _SC_AGG[nc](hrows, ex.reshape(EPAD, 16), rinv,
                        src2c, dst2c, zf)
    f2 = fpart.transpose(0, 2, 1, 3).reshape(2, NPAD, copad)[:, :N]
    c2 = cpart[:, :N, :16]
    featp, coordo = _combine(f2[0], f2[1], c2[0], c2[1], coordp, maskp)
    return coordo, featp[:, :cout]


def kernel(data, edge_idx, bd_mask, poly_mesh, lin_w, lin_b, W1, b1, att_src1, att_dst1, W2, b2, att_src2, att_dst2, W3, b3, att_src3, att_dst3, W4, b4, att_src4, att_dst4):
    src = edge_idx[0]
    dst = edge_idx[1]
    srcp = jnp.concatenate([src, jnp.zeros((EPAD - E,), jnp.int32)])
    dstp = jnp.concatenate([dst, jnp.full((EPAD - E,), NPAD - 1, jnp.int32)])
    src3 = srcp.reshape(EPAD // 64, 64)
    dst3 = dstp.reshape(EPAD // 64, 64)
    src2c = srcp.reshape(EPAD // 128, 128)
    dst2c = dstp.reshape(EPAD // 128, 128)
    zf = jnp.zeros((ROWS_PT, 128), jnp.float32)

    coords = data[:, 0:2]
    coordp = jnp.pad(coords, ((0, 0), (0, 14)))
    maskp = jnp.broadcast_to(bd_mask.astype(jnp.float32)[:, None], (N, 16))

    lin1 = _mm_selu(data, lin_w, lin_b)
    t = jnp.concatenate([coords, lin1], axis=1)

    c1, f1 = _layer(t, coordp, maskp, src3, dst3, src2c, dst2c, zf,
                    W1, b1, att_src1, att_dst1, 508, 512)
    t = jnp.concatenate([c1[:, :2], coords, f1], axis=1)
    c2, f2 = _layer(t, c1, maskp, src3, dst3, src2c, dst2c, zf,
                    W2, b2, att_src2, att_dst2, 250, 256)
    t = jnp.concatenate([c2[:, :2], c1[:, :2], coords, f2], axis=1)
    c3, f3 = _layer(t, c2, maskp, src3, dst3, src2c, dst2c, zf,
                    W3, b3, att_src3, att_dst3, 120, 128)
    t = jnp.concatenate([c3[:, :2], c2[:, :2], c1[:, :2], coords, f3], axis=1)
    c4, f4 = _layer(t, c3, maskp, src3, dst3, src2c, dst2c, zf,
                    W4, b4, att_src4, att_dst4, 20, 128)
    return c4[:, :2]


# speedup vs baseline: 10.0894x; 1.2382x over previous
import functools

import jax
import jax.numpy as jnp
from jax import lax
from jax.experimental import pallas as pl
from jax.experimental.pallas import tpu as pltpu
from jax.experimental.pallas import tpu_sc as plsc

N = 10000
E = 160000
HEADS = 6
NPAD = 10240
EPAD = 163840
NTILES = 32
EPT = EPAD // NTILES
NB1 = EPT // 128
NB2 = EPT // 64
ROWS_PT = NPAD // 16

_SC_MESH = plsc.VectorSubcoreMesh(core_axis_name="c", subcore_axis_name="s",
                                  num_cores=2, num_subcores=16)


def _selu(v):
    return 1.0507009873554805 * jnp.where(
        v > 0, v, 1.6732632423543772 * (jnp.exp(jnp.minimum(v, 0.0)) - 1.0))


def _mm_selu(x, w, b):
    n, k = x.shape
    m = w.shape[1]
    bn = 1000

    def body(x_ref, w_ref, b_ref, o_ref):
        h = jnp.dot(x_ref[...], w_ref[...], preferred_element_type=jnp.float32)
        o_ref[...] = _selu(h + b_ref[...])

    return pl.pallas_call(
        body,
        grid=(n // bn,),
        in_specs=[
            pl.BlockSpec((bn, k), lambda i: (i, 0)),
            pl.BlockSpec((k, m), lambda i: (0, 0)),
            pl.BlockSpec((1, m), lambda i: (0, 0)),
        ],
        out_specs=pl.BlockSpec((bn, m), lambda i: (i, 0)),
        out_shape=jax.ShapeDtypeStruct((n, m), jnp.float32),
    )(x, w, b.reshape(1, m))


def _gat_dense(x, wp, bp, amat):
    n, k = x.shape
    m = wp.shape[1]
    bn = 1000

    def body(x_ref, w_ref, b_ref, a_ref, h_ref, asd_ref, mx_ref):
        i = pl.program_id(0)
        h = jnp.dot(x_ref[...], w_ref[...], preferred_element_type=jnp.float32)
        h = h + b_ref[...]
        h_ref[...] = h
        asd = jnp.dot(h, a_ref[...], preferred_element_type=jnp.float32)
        asd_ref[...] = asd

        @pl.when(i == 0)
        def _():
            mx_ref[...] = jnp.full((8, 128), -1e30, jnp.float32)

        mxb = jnp.max(asd, axis=0, keepdims=True)
        mx_ref[...] = jnp.maximum(mx_ref[...], jnp.broadcast_to(mxb, (8, 128)))

    return pl.pallas_call(
        body,
        grid=(n // bn,),
        in_specs=[
            pl.BlockSpec((bn, k), lambda i: (i, 0)),
            pl.BlockSpec((k, m), lambda i: (0, 0)),
            pl.BlockSpec((1, m), lambda i: (0, 0)),
            pl.BlockSpec((m, 128), lambda i: (0, 0)),
        ],
        out_specs=[
            pl.BlockSpec((bn, m), lambda i: (i, 0)),
            pl.BlockSpec((bn, 128), lambda i: (i, 0)),
            pl.BlockSpec((8, 128), lambda i: (0, 0)),
        ],
        out_shape=[
            jax.ShapeDtypeStruct((n, m), jnp.float32),
            jax.ShapeDtypeStruct((n, 128), jnp.float32),
            jax.ShapeDtypeStruct((8, 128), jnp.float32),
        ],
    )(x, wp, bp.reshape(1, m), amat)


def _recip(s0, s1):
    n, m = s0.shape
    bn = 2048

    def body(a_ref, b_ref, o_ref):
        o_ref[...] = (1.0 / HEADS) / (a_ref[...] + b_ref[...] + 1e-16)

    return pl.pallas_call(
        body,
        grid=(n // bn,),
        in_specs=[
            pl.BlockSpec((bn, m), lambda i: (i, 0)),
            pl.BlockSpec((bn, m), lambda i: (i, 0)),
        ],
        out_specs=pl.BlockSpec((bn, m), lambda i: (i, 0)),
        out_shape=jax.ShapeDtypeStruct((n, m), jnp.float32),
    )(s0, s1)


def _combine(f0, f1, c0, c1, coordp, maskp):
    n, m = f0.shape
    bn = 2000

    def body(f0r, f1r, c0r, c1r, cpr, mkr, fo, co):
        fo[...] = _selu(f0r[...] + f1r[...])
        mk = mkr[...]
        co[...] = mk * cpr[...] + (1.0 - mk) * (c0r[...] + c1r[...])

    return pl.pallas_call(
        body,
        grid=(n // bn,),
        in_specs=[
            pl.BlockSpec((bn, m), lambda i: (i, 0)),
            pl.BlockSpec((bn, m), lambda i: (i, 0)),
            pl.BlockSpec((bn, 16), lambda i: (i, 0)),
            pl.BlockSpec((bn, 16), lambda i: (i, 0)),
            pl.BlockSpec((bn, 16), lambda i: (i, 0)),
            pl.BlockSpec((bn, 16), lambda i: (i, 0)),
        ],
        out_specs=[
            pl.BlockSpec((bn, m), lambda i: (i, 0)),
            pl.BlockSpec((bn, 16), lambda i: (i, 0)),
        ],
        out_shape=[
            jax.ShapeDtypeStruct((n, m), jnp.float32),
            jax.ShapeDtypeStruct((n, 16), jnp.float32),
        ],
    )(f0, f1, c0, c1, coordp, maskp)


def _tile_id():
    return lax.axis_index("c") * 16 + lax.axis_index("s")


def _zero_rows(zb, shared, srows):
    nz = zb.shape[0]
    for k2 in range(ROWS_PT // nz):
        pltpu.sync_copy(zb, shared.at[pl.ds(srows + k2 * nz, nz)])


NB1B = EPT // 64


def _zero_high_cols(buf):
    def zrow(i, _):
        for g in range(1, 8):
            buf[i, pl.ds(g * 16, 16)] = jnp.zeros((16,), jnp.float32)
        return 0

    lax.fori_loop(0, buf.shape[0], zrow, 0)


NBE = EPAD // 16
NBT = EPT // 16
NIT = NBT // 2
PKR = EPT // 128


def _edge_vecs(srcb, dstb, bid):
    r = bid // 8
    co = (bid % 8) * 16
    return srcb[r, pl.ds(co, 16)], dstb[r, pl.ds(co, 16)]


def _sc_scores(asrc_d, adst_d, mvec, src2c, dst2c, zf):
    def body(asrc_ref, adst_ref, mv_ref, src_ref, dst_ref, z_ref,
             ex_out, s_out, srcb, dstb, aga, dga, exwa, ex16a,
             agb, dgb, exwb, ex16b, mvb, s_sh,
             gsema, gsemb, ssema, ssemb, wsema, wsemb):
        core = lax.axis_index("c")
        sub = lax.axis_index("s")
        wid = _tile_id()
        pbase = wid * PKR
        ebase = wid * EPT
        pltpu.sync_copy(src_ref.at[pl.ds(pbase, PKR)], srcb)
        pltpu.sync_copy(dst_ref.at[pl.ds(pbase, PKR)], dstb)
        pltpu.sync_copy(mv_ref, mvb)
        pltpu.sync_copy(z_ref, s_sh.at[pl.ds(sub * ROWS_PT, ROWS_PT)])
        _zero_high_cols(exwa)
        _zero_high_cols(exwb)
        plsc.subcore_barrier()

        mvv = mvb[...]

        def fire(bid, agx, dgx, gsemx):
            sv, dv = _edge_vecs(srcb, dstb, bid)
            pltpu.async_copy(asrc_ref.at[sv], agx, gsemx)
            pltpu.async_copy(adst_ref.at[dv], dgx, gsemx)

        def drain_g(agx, dgx, gsemx):
            pltpu.make_async_copy(asrc_ref.at[pl.ds(0, 16)], agx,
                                  gsemx).wait()
            pltpu.make_async_copy(adst_ref.at[pl.ds(0, 16)], dgx,
                                  gsemx).wait()

        def drain_w(exwx, ex16x, ssemx, wsemx):
            pltpu.make_async_copy(exwx, s_sh.at[pl.ds(0, 16)], ssemx).wait()
            pltpu.make_async_copy(ex16x, ex_out.at[pl.ds(0, 16)],
                                  wsemx).wait()

        def compute(bid, agx, dgx, exwx, ex16x, ssemx, wsemx):
            def row(i, _):
                s16 = pl.ds(0, 16)
                e = agx[i, s16] + dgx[i, s16]
                e = jnp.where(e > 0, e, 0.2 * e)
                v = jnp.exp(e - mvv)
                exwx[i, s16] = v
                ex16x[i, :] = v
                return 0

            lax.fori_loop(0, 16, row, 0)
            _, dv = _edge_vecs(srcb, dstb, bid)
            pltpu.async_copy(ex16x, ex_out.at[pl.ds(ebase + bid * 16, 16)],
                             wsemx)
            pltpu.async_copy(exwx, s_sh.at[dv], ssemx, add=True)

        fire(0, aga, dga, gsema)

        def it(t, _):
            fire(2 * t + 1, agb, dgb, gsemb)
            drain_g(aga, dga, gsema)

            @pl.when(t > 0)
            def _():
                drain_w(exwa, ex16a, ssema, wsema)

            compute(2 * t, aga, dga, exwa, ex16a, ssema, wsema)

            @pl.when(t < NIT - 1)
            def _():
                fire(2 * t + 2, aga, dga, gsema)

            drain_g(agb, dgb, gsemb)

            @pl.when(t > 0)
            def _():
                drain_w(exwb, ex16b, ssemb, wsemb)

            compute(2 * t + 1, agb, dgb, exwb, ex16b, ssemb, wsemb)
            return 0

        lax.fori_loop(0, NIT, it, 0)
        drain_w(exwa, ex16a, ssema, wsema)
        drain_w(exwb, ex16b, ssemb, wsemb)
        plsc.subcore_barrier()
        pltpu.sync_copy(s_sh.at[pl.ds(sub * ROWS_PT, ROWS_PT)],
                        s_out.at[core, pl.ds(sub * ROWS_PT, ROWS_PT)])

    return pl.kernel(
        body,
        out_type=[
            jax.ShapeDtypeStruct((EPAD, 16), jnp.float32),
            jax.ShapeDtypeStruct((2, NPAD, 128), jnp.float32),
        ],
        mesh=_SC_MESH,
        scratch_types=[
            pltpu.VMEM((PKR, 128), jnp.int32),
            pltpu.VMEM((PKR, 128), jnp.int32),
            pltpu.VMEM((16, 128), jnp.float32),
            pltpu.VMEM((16, 128), jnp.float32),
            pltpu.VMEM((16, 128), jnp.float32),
            pltpu.VMEM((16, 16), jnp.float32),
            pltpu.VMEM((16, 128), jnp.float32),
            pltpu.VMEM((16, 128), jnp.float32),
            pltpu.VMEM((16, 128), jnp.float32),
            pltpu.VMEM((16, 16), jnp.float32),
            pltpu.VMEM((16,), jnp.float32),
            pltpu.VMEM_SHARED((NPAD, 128), jnp.float32),
            pltpu.SemaphoreType.DMA,
            pltpu.SemaphoreType.DMA,
            pltpu.SemaphoreType.DMA,
            pltpu.SemaphoreType.DMA,
            pltpu.SemaphoreType.DMA,
            pltpu.SemaphoreType.DMA,
        ],
    )(asrc_d, adst_d, mvec, src2c, dst2c, zf)


def _sc_coords(ex, rinv, coord_g, src2c, dst2c, zf):
    def body(ex_ref, rinv_ref, coord_ref, src_ref, dst_ref, z_ref,
             c_out, srcb, dstb, exga, riga, coga, cmwa,
             exgb, rigb, cogb, cmwb, c_sh,
             gsema, gsemb, ssema, ssemb):
        core = lax.axis_index("c")
        sub = lax.axis_index("s")
        wid = _tile_id()
        pbase = wid * PKR
        ebase = wid * EPT
        pltpu.sync_copy(src_ref.at[pl.ds(pbase, PKR)], srcb)
        pltpu.sync_copy(dst_ref.at[pl.ds(pbase, PKR)], dstb)
        pltpu.sync_copy(z_ref, c_sh.at[pl.ds(sub * ROWS_PT, ROWS_PT)])
        _zero_high_cols(cmwa)
        _zero_high_cols(cmwb)
        plsc.subcore_barrier()

        def fire(bid, exgx, rigx, cogx, gsemx):
            sv, dv = _edge_vecs(srcb, dstb, bid)
            pltpu.async_copy(ex_ref.at[pl.ds(ebase + bid * 16, 16)], exgx,
                             gsemx)
            pltpu.async_copy(rinv_ref.at[dv], rigx, gsemx)
            pltpu.async_copy(coord_ref.at[sv], cogx, gsemx)

        def drain_g(exgx, rigx, cogx, gsemx):
            pltpu.make_async_copy(ex_ref.at[pl.ds(0, 16)], exgx,
                                  gsemx).wait()
            pltpu.make_async_copy(rinv_ref.at[pl.ds(0, 16)], rigx,
                                  gsemx).wait()
            pltpu.make_async_copy(coord_ref.at[pl.ds(0, 16)], cogx,
                                  gsemx).wait()

        def drain_w(cmwx, ssemx):
            pltpu.make_async_copy(cmwx, c_sh.at[pl.ds(0, 16)], ssemx).wait()

        def compute(bid, exgx, rigx, cogx, cmwx, ssemx):
            def row(i, _):
                s16 = pl.ds(0, 16)
                av = exgx[i, :] * rigx[i, s16]
                am = av[0] + av[1] + av[2] + av[3] + av[4] + av[5]
                cmwx[i, s16] = am * cogx[i, s16]
                return 0

            lax.fori_loop(0, 16, row, 0)
            _, dv = _edge_vecs(srcb, dstb, bid)
            pltpu.async_copy(cmwx, c_sh.at[dv], ssemx, add=True)

        fire(0, exga, riga, coga, gsema)

        def it(t, _):
            fire(2 * t + 1, exgb, rigb, cogb, gsemb)
            drain_g(exga, riga, coga, gsema)

            @pl.when(t > 0)
            def _():
                drain_w(cmwa, ssema)

            compute(2 * t, exga, riga, coga, cmwa, ssema)

            @pl.when(t < NIT - 1)
            def _():
                fire(2 * t + 2, exga, riga, coga, gsema)

            drain_g(exgb, rigb, cogb, gsemb)

            @pl.when(t > 0)
            def _():
                drain_w(cmwb, ssemb)

            compute(2 * t + 1, exgb, rigb, cogb, cmwb, ssemb)
            return 0

        lax.fori_loop(0, NIT, it, 0)
        drain_w(cmwa, ssema)
        drain_w(cmwb, ssemb)
        plsc.subcore_barrier()
        pltpu.sync_copy(c_sh.at[pl.ds(sub * ROWS_PT, ROWS_PT)],
                        c_out.at[core, pl.ds(sub * ROWS_PT, ROWS_PT)])

    return pl.kernel(
        body,
        out_type=jax.ShapeDtypeStruct((2, NPAD, 128), jnp.float32),
        mesh=_SC_MESH,
        scratch_types=[
            pltpu.VMEM((PKR, 128), jnp.int32),
            pltpu.VMEM((PKR, 128), jnp.int32),
            pltpu.VMEM((16, 16), jnp.float32),
            pltpu.VMEM((16, 128), jnp.float32),
            pltpu.VMEM((16, 128), jnp.float32),
            pltpu.VMEM((16, 128), jnp.float32),
            pltpu.VMEM((16, 16), jnp.float32),
            pltpu.VMEM((16, 128), jnp.float32),
            pltpu.VMEM((16, 128), jnp.float32),
            pltpu.VMEM((16, 128), jnp.float32),
            pltpu.VMEM_SHARED((NPAD, 128), jnp.float32),
            pltpu.SemaphoreType.DMA,
            pltpu.SemaphoreType.DMA,
            pltpu.SemaphoreType.DMA,
            pltpu.SemaphoreType.DMA,
        ],
    )(ex, rinv, coord_g, src2c, dst2c, zf)


NB2 = EPT // 16
NI2 = NB2 // 2


def _make_sc_agg(nc):
    def body(hrows, ex_ref, rinv_ref, src_ref, dst_ref, z_ref,
             f_out, srcb, dstb, exga, riga, rowsa,
             exgb, rigb, rowsb, msga, msgb, f_sh,
             gsema, gsemb, ssema, ssemb):
        core = lax.axis_index("c")
        sub = lax.axis_index("s")
        wid = _tile_id()
        pbase = wid * (EPT // 128)
        ebase = wid * EPT
        pltpu.sync_copy(src_ref.at[pl.ds(pbase, EPT // 128)], srcb)
        pltpu.sync_copy(dst_ref.at[pl.ds(pbase, EPT // 128)], dstb)

        def edge_vecs(bid):
            r = bid // 8
            co = (bid % 8) * 16
            return srcb[r, pl.ds(co, 16)], dstb[r, pl.ds(co, 16)]

        def fire(bid, exgx, rigx, rowsx, gsemx, ci):
            sv, dv = edge_vecs(bid)
            pltpu.async_copy(ex_ref.at[pl.ds(ebase + bid * 16, 16)], exgx,
                             gsemx)
            pltpu.async_copy(rinv_ref.at[dv], rigx, gsemx)
            for hh in range(HEADS):
                iv = sv * (HEADS * nc) + (hh * nc + ci)
                pltpu.async_copy(hrows.at[iv], rowsx.at[hh], gsemx)

        def drain(exgx, rigx, rowsx, gsemx):
            pltpu.make_async_copy(ex_ref.at[pl.ds(0, 16)], exgx,
                                  gsemx).wait()
            pltpu.make_async_copy(rinv_ref.at[pl.ds(0, 16)], rigx,
                                  gsemx).wait()
            for hh in range(HEADS):
                pltpu.make_async_copy(hrows.at[pl.ds(0, 16)], rowsx.at[hh],
                                      gsemx).wait()

        def drain_s(msgx, ssemx):
            pltpu.make_async_copy(msgx, f_sh.at[pl.ds(0, 16)], ssemx).wait()

        def compute(bid, exgx, rigx, rowsx, msgx, ssemx):
            def erow(i, _):
                av = exgx[i, :] * rigx[i, pl.ds(0, 16)]
                a0 = av[0]
                a1 = av[1]
                a2 = av[2]
                a3 = av[3]
                a4 = av[4]
                a5 = av[5]
                for g in range(8):
                    sl = pl.ds(g * 16, 16)
                    v = a0 * rowsx[0, i, sl]
                    v = v + a1 * rowsx[1, i, sl]
                    v = v + a2 * rowsx[2, i, sl]
                    v = v + a3 * rowsx[3, i, sl]
                    v = v + a4 * rowsx[4, i, sl]
                    v = v + a5 * rowsx[5, i, sl]
                    msgx[i, sl] = v
                return 0

            lax.fori_loop(0, 16, erow, 0)
            _, dv = edge_vecs(bid)
            pltpu.async_copy(msgx, f_sh.at[dv], ssemx, add=True)

        for ci in range(nc):
            pltpu.sync_copy(z_ref, f_sh.at[pl.ds(sub * ROWS_PT, ROWS_PT)])
            plsc.subcore_barrier()
            fire(0, exga, riga, rowsa, gsema, ci)

            def it(t, _):
                fire(2 * t + 1, exgb, rigb, rowsb, gsemb, ci)
                drain(exga, riga, rowsa, gsema)

                @pl.when(t > 0)
                def _():
                    drain_s(msga, ssema)

                compute(2 * t, exga, riga, rowsa, msga, ssema)

                @pl.when(t < NI2 - 1)
                def _():
                    fire(2 * t + 2, exga, riga, rowsa, gsema, ci)

                drain(exgb, rigb, rowsb, gsemb)

                @pl.when(t > 0)
                def _():
                    drain_s(msgb, ssemb)

                compute(2 * t + 1, exgb, rigb, rowsb, msgb, ssemb)
                return 0

            lax.fori_loop(0, NI2, it, 0)
            drain_s(msga, ssema)
            drain_s(msgb, ssemb)
            plsc.subcore_barrier()
            pltpu.sync_copy(
                f_sh.at[pl.ds(sub * ROWS_PT, ROWS_PT)],
                f_out.at[core, ci, pl.ds(sub * ROWS_PT, ROWS_PT)])
            plsc.subcore_barrier()

    return pl.kernel(
        body,
        out_type=jax.ShapeDtypeStruct((2, nc, NPAD, 128), jnp.float32),
        mesh=_SC_MESH,
        scratch_types=[
            pltpu.VMEM((EPT // 128, 128), jnp.int32),
            pltpu.VMEM((EPT // 128, 128), jnp.int32),
            pltpu.VMEM((16, 16), jnp.float32),
            pltpu.VMEM((16, 128), jnp.float32),
            pltpu.VMEM((HEADS, 16, 128), jnp.float32),
            pltpu.VMEM((16, 16), jnp.float32),
            pltpu.VMEM((16, 128), jnp.float32),
            pltpu.VMEM((HEADS, 16, 128), jnp.float32),
            pltpu.VMEM((16, 128), jnp.float32),
            pltpu.VMEM((16, 128), jnp.float32),
            pltpu.VMEM_SHARED((NPAD, 128), jnp.float32),
            pltpu.SemaphoreType.DMA,
            pltpu.SemaphoreType.DMA,
            pltpu.SemaphoreType.DMA,
            pltpu.SemaphoreType.DMA,
        ],
    )


_SC_AGG = {1: _make_sc_agg(1), 2: _make_sc_agg(2), 4: _make_sc_agg(4)}


def _pad_layer_params(W, b, a_s, a_d, cin, cout, copad):
    hw = W.reshape(cin, HEADS, cout)
    wp = jnp.pad(hw, ((0, 0), (0, 0), (0, copad - cout))).reshape(cin, HEADS * copad)
    hb = b.reshape(HEADS, cout)
    bp = jnp.pad(hb, ((0, 0), (0, copad - cout))).reshape(HEADS * copad)
    asp = jnp.pad(a_s, ((0, 0), (0, copad - cout)))
    adp = jnp.pad(a_d, ((0, 0), (0, copad - cout)))
    eye = jnp.eye(HEADS, 8, dtype=jnp.float32)
    ablk_s = asp[:, :, None] * eye[:, None, :]
    ablk_d = adp[:, :, None] * eye[:, None, :]
    amat = jnp.concatenate([ablk_s, ablk_d], axis=2).reshape(HEADS * copad, 16)
    amat = jnp.pad(amat, ((0, 0), (0, 112)))
    return wp, bp, amat


def _layer(x, coordp, maskp, src2c, dst2c, zf,
           W, b, a_s, a_d, cout, copad):
    cin = x.shape[1]
    wp, bp, amat = _pad_layer_params(W, b, a_s, a_d, cin, cout, copad)
    h, asd, mx = _gat_dense(x, wp, bp, amat)
    m8 = mx[0, :8] + mx[0, 8:16]
    mvec = jnp.concatenate([m8, m8])
    asrc_d = jnp.pad(jnp.concatenate([asd[:, :8], asd[:, :8]], axis=1),
                     ((0, NPAD - N), (0, 112)))
    adst_d = jnp.pad(jnp.concatenate([asd[:, 8:16], asd[:, 8:16]], axis=1),
                     ((0, NPAD - N), (0, 112)))
    coord_g = jnp.pad(coordp, ((0, NPAD - N), (0, 112)))

    ex, spart = _sc_scores(asrc_d, adst_d, mvec, src2c, dst2c, zf)
    rinv = jnp.pad(_recip(spart[0, :, :16], spart[1, :, :16]),
                   ((0, 0), (0, 112)))

    cpart = _sc_coords(ex, rinv, coord_g, src2c, dst2c, zf)
    nc = copad // 128
    hrows = h.reshape(N * HEADS * nc, 128)
    fpart = _SC_AGG[nc](hrows, ex, rinv,
                        src2c, dst2c, zf)
    f2 = fpart.transpose(0, 2, 1, 3).reshape(2, NPAD, copad)[:, :N]
    c2 = cpart[:, :N, :16]
    featp, coordo = _combine(f2[0], f2[1], c2[0], c2[1], coordp, maskp)
    return coordo, featp[:, :cout]


def kernel(data, edge_idx, bd_mask, poly_mesh, lin_w, lin_b, W1, b1, att_src1, att_dst1, W2, b2, att_src2, att_dst2, W3, b3, att_src3, att_dst3, W4, b4, att_src4, att_dst4):
    src = edge_idx[0]
    dst = edge_idx[1]
    srcp = jnp.concatenate([src, jnp.zeros((EPAD - E,), jnp.int32)])
    dstp = jnp.concatenate([dst, jnp.full((EPAD - E,), NPAD - 1, jnp.int32)])
    src2c = srcp.reshape(EPAD // 128, 128)
    dst2c = dstp.reshape(EPAD // 128, 128)
    zf = jnp.zeros((ROWS_PT, 128), jnp.float32)

    coords = data[:, 0:2]
    coordp = jnp.pad(coords, ((0, 0), (0, 14)))
    maskp = jnp.broadcast_to(bd_mask.astype(jnp.float32)[:, None], (N, 16))

    lin1 = _mm_selu(data, lin_w, lin_b)
    t = jnp.concatenate([coords, lin1], axis=1)

    c1, f1 = _layer(t, coordp, maskp, src2c, dst2c, zf,
                    W1, b1, att_src1, att_dst1, 508, 512)
    t = jnp.concatenate([c1[:, :2], coords, f1], axis=1)
    c2, f2 = _layer(t, c1, maskp, src2c, dst2c, zf,
                    W2, b2, att_src2, att_dst2, 250, 256)
    t = jnp.concatenate([c2[:, :2], c1[:, :2], coords, f2], axis=1)
    c3, f3 = _layer(t, c2, maskp, src2c, dst2c, zf,
                    W3, b3, att_src3, att_dst3, 120, 128)
    t = jnp.concatenate([c3[:, :2], c2[:, :2], c1[:, :2], coords, f3], axis=1)
    c4, f4 = _layer(t, c3, maskp, src2c, dst2c, zf,
                    W4, b4, att_src4, att_dst4, 20, 128)
    return c4[:, :2]
